# initial kernel scaffold (unmeasured)
import jax
import jax.numpy as jnp
from jax import lax
from jax.experimental import pallas as pl
from jax.experimental.pallas import tpu as pltpu


def kernel(
    x,
):
    def body(*refs):
        pass

    out_shape = jax.ShapeDtypeStruct(..., jnp.float32)
    return pl.pallas_call(body, out_shape=out_shape)(...)


# baseline (device time: 13798 ns/iter reference)
import jax
import jax.numpy as jnp
from jax import lax
from jax.experimental import pallas as pl
from jax.experimental.pallas import tpu as pltpu

N_DEV = 32


def kernel(x):
    m_per, n = x.shape

    def body(x_ref, out_ref, comm_ref, send_sems, recv_sems):
        my_pos = lax.axis_index("i")

        xv = x_ref[:, :]
        row = lax.broadcasted_iota(jnp.int32, xv.shape, 0)
        m0 = jnp.max(xv, axis=0)
        cand = jnp.where(xv == m0[None, :], row, jnp.int32(2**30))
        li = jnp.min(cand, axis=0)
        gi = (li + my_pos * m_per).astype(jnp.float32)
        comm_ref[0, 0, :] = m0
        comm_ref[0, 1, :] = gi

        barrier_sem = pltpu.get_barrier_semaphore()
        for d in range(1, N_DEV):
            pl.semaphore_signal(
                barrier_sem, inc=1,
                device_id=((my_pos + d) % N_DEV,),
                device_id_type=pl.DeviceIdType.MESH,
            )
        pl.semaphore_wait(barrier_sem, N_DEV - 1)

        rdmas = []
        for d in range(1, N_DEV):
            rdma = pltpu.make_async_remote_copy(
                src_ref=comm_ref.at[0],
                dst_ref=comm_ref.at[d],
                send_sem=send_sems.at[d],
                recv_sem=recv_sems.at[d],
                device_id=((my_pos + d) % N_DEV,),
                device_id_type=pl.DeviceIdType.MESH,
            )
            rdma.start()
            rdmas.append(rdma)
        for rdma in rdmas:
            rdma.wait()

        vals = comm_ref[:, 0, :]
        idxs = comm_ref[:, 1, :]
        best = jnp.max(vals, axis=0)
        pick = jnp.where(vals == best[None, :], idxs, jnp.float32(jnp.inf))
        best_idx = jnp.min(pick, axis=0)
        out_ref[0, :] = best
        out_ref[1, :] = best_idx

    return pl.pallas_call(
        body,
        out_shape=jax.ShapeDtypeStruct((2, n), jnp.float32),
        in_specs=[pl.BlockSpec(memory_space=pltpu.VMEM)],
        out_specs=pl.BlockSpec(memory_space=pltpu.VMEM),
        scratch_shapes=[
            pltpu.VMEM((N_DEV, 2, n), jnp.float32),
            pltpu.SemaphoreType.DMA((N_DEV,)),
            pltpu.SemaphoreType.DMA((N_DEV,)),
        ],
        compiler_params=pltpu.CompilerParams(collective_id=0),
    )(x)


# device time: 13495 ns/iter; 1.0225x vs baseline; 1.0225x over previous
import jax
import jax.numpy as jnp
from jax import lax
from jax.experimental import pallas as pl
from jax.experimental.pallas import tpu as pltpu

N_DEV = 32


def kernel(x):
    m_per, n = x.shape

    def body(x_ref, out_ref, comm_ref, send_sems, recv_sems):
        my_pos = lax.axis_index("i")

        barrier_sem = pltpu.get_barrier_semaphore()
        for d in range(1, N_DEV):
            pl.semaphore_signal(
                barrier_sem, inc=1,
                device_id=((my_pos + d) % N_DEV,),
                device_id_type=pl.DeviceIdType.MESH,
            )

        xv = x_ref[:, :]
        row = lax.broadcasted_iota(jnp.int32, xv.shape, 0)
        m0 = jnp.max(xv, axis=0)
        cand = jnp.where(xv == m0[None, :], row, jnp.int32(2**30))
        li = jnp.min(cand, axis=0)
        gi = (li + my_pos * m_per).astype(jnp.float32)
        comm_ref[0, 0, :] = m0
        comm_ref[0, 1, :] = gi

        pl.semaphore_wait(barrier_sem, N_DEV - 1)

        rdmas = []
        for d in range(1, N_DEV):
            rdma = pltpu.make_async_remote_copy(
                src_ref=comm_ref.at[0],
                dst_ref=comm_ref.at[d],
                send_sem=send_sems.at[d],
                recv_sem=recv_sems.at[d],
                device_id=((my_pos + d) % N_DEV,),
                device_id_type=pl.DeviceIdType.MESH,
            )
            rdma.start()
            rdmas.append(rdma)
        for rdma in rdmas:
            rdma.wait()

        vals = comm_ref[:, 0, :]
        idxs = comm_ref[:, 1, :]
        best = jnp.max(vals, axis=0)
        pick = jnp.where(vals == best[None, :], idxs, jnp.float32(jnp.inf))
        best_idx = jnp.min(pick, axis=0)
        out_ref[0, :] = best
        out_ref[1, :] = best_idx

    return pl.pallas_call(
        body,
        out_shape=jax.ShapeDtypeStruct((2, n), jnp.float32),
        in_specs=[pl.BlockSpec(memory_space=pltpu.VMEM)],
        out_specs=pl.BlockSpec(memory_space=pltpu.VMEM),
        scratch_shapes=[
            pltpu.VMEM((N_DEV, 2, n), jnp.float32),
            pltpu.SemaphoreType.DMA((N_DEV,)),
            pltpu.SemaphoreType.DMA((N_DEV,)),
        ],
        compiler_params=pltpu.CompilerParams(collective_id=0),
    )(x)


# device time: 12493 ns/iter; 1.1045x vs baseline; 1.0802x over previous
import jax
import jax.numpy as jnp
from jax import lax
from jax.experimental import pallas as pl
from jax.experimental.pallas import tpu as pltpu

N_DEV = 32


def kernel(x):
    m_per, n = x.shape

    def body(x_ref, out_ref, comm_ref, send_sems, recv_sems):
        my_pos = lax.axis_index("i")

        barrier_sem = pltpu.get_barrier_semaphore()
        for d in range(1, N_DEV):
            pl.semaphore_signal(
                barrier_sem, inc=1,
                device_id=((my_pos + d) % N_DEV,),
                device_id_type=pl.DeviceIdType.MESH,
            )

        xv = x_ref[:, :]
        row = lax.broadcasted_iota(jnp.int32, xv.shape, 0)
        m0 = jnp.max(xv, axis=0)
        cand = jnp.where(xv == m0[None, :], row, jnp.int32(2**30))
        li = jnp.min(cand, axis=0)
        gi = (li + my_pos * m_per).astype(jnp.float32)
        comm_ref[0, 0, :] = m0
        comm_ref[0, 1, :] = gi

        pl.semaphore_wait(barrier_sem, N_DEV - 1)

        rdmas = []
        for d in range(1, N_DEV):
            rdma = pltpu.make_async_remote_copy(
                src_ref=comm_ref.at[0],
                dst_ref=comm_ref.at[d],
                send_sem=send_sems.at[d],
                recv_sem=recv_sems.at[d],
                device_id=((my_pos + d) % N_DEV,),
                device_id_type=pl.DeviceIdType.MESH,
            )
            rdma.start()
            rdmas.append(rdma)

        best = None
        best_idx = None
        CHUNK = 8
        for lo in range(0, N_DEV, CHUNK):
            hi = min(lo + CHUNK, N_DEV)
            for d in range(max(lo, 1), hi):
                rdmas[d - 1].wait()
            vals = comm_ref[lo:hi, 0, :]
            idxs = comm_ref[lo:hi, 1, :]
            m = jnp.max(vals, axis=0)
            pick = jnp.where(vals == m[None, :], idxs, jnp.float32(jnp.inf))
            mi = jnp.min(pick, axis=0)
            if best is None:
                best, best_idx = m, mi
            else:
                take = (m > best) | ((m == best) & (mi < best_idx))
                best = jnp.where(take, m, best)
                best_idx = jnp.where(take, mi, best_idx)
        out_ref[0, :] = best
        out_ref[1, :] = best_idx

    return pl.pallas_call(
        body,
        out_shape=jax.ShapeDtypeStruct((2, n), jnp.float32),
        in_specs=[pl.BlockSpec(memory_space=pltpu.VMEM)],
        out_specs=pl.BlockSpec(memory_space=pltpu.VMEM),
        scratch_shapes=[
            pltpu.VMEM((N_DEV, 2, n), jnp.float32),
            pltpu.SemaphoreType.DMA((N_DEV,)),
            pltpu.SemaphoreType.DMA((N_DEV,)),
        ],
        compiler_params=pltpu.CompilerParams(collective_id=0),
    )(x)
